# baseline (speedup 1.0000x reference)
import jax
import jax.numpy as jnp
from jax.experimental import pallas as pl
from jax.experimental.pallas import tpu as pltpu

_N = 250
_L = 128
_BN = 6784


def _clamp01(x):
    return jnp.minimum(jnp.maximum(x, 0.0), 1.0)


def _tel_kernel(t_ref, xt_ref, wv_ref, a_ref, b_ref, out_ref, wf_ref):
    @pl.when(pl.program_id(0) == 0)
    def _prep():
        a = jax.nn.sigmoid(a_ref[0, 0:1, :])
        b = jax.nn.sigmoid(b_ref[0, 0:1, :])

        w_t = wv_ref[:, t_ref[0], :]

        wh = w_t[:_N]
        wh = jnp.exp(wh - jnp.max(wh, axis=0, keepdims=True))
        wh = wh * (a / jnp.sum(wh, axis=0, keepdims=True))

        ws = w_t[_N:]
        ws = jnp.exp(ws - jnp.max(ws, axis=0, keepdims=True))
        ws = ws * (b / jnp.sum(ws, axis=0, keepdims=True))

        c = 1.0 - _clamp01(a + b)

        w_full = jnp.concatenate([wh, ws, c], axis=0)
        wf_ref[...] = w_full.astype(jnp.bfloat16)

    acc = jax.lax.dot_general(xt_ref[...].astype(jnp.bfloat16),
                              wf_ref[...],
                              (((0,), (0,)), ((), ())),
                              preferred_element_type=jnp.float32)
    out_ref[...] = _clamp01(acc)


def kernel(input, input_all, all_states, t, entity2id, flag, w, w_inv,
           weight, alpha, beta):
    n = _N
    n_ent = entity2id.shape[0]
    m = input.shape[0]
    k = input.shape[1]
    nb = m // n_ent
    nt = w.shape[0]

    xt = input.T
    wv = w.transpose(2, 0, 1)
    t_arr = jnp.asarray(t, jnp.int32).reshape(1)

    grid_spec = pltpu.PrefetchScalarGridSpec(
        num_scalar_prefetch=1,
        grid=(pl.cdiv(m, _BN),),
        in_specs=[
            pl.BlockSpec((k, _BN), lambda i, t_pre: (0, i)),
            pl.BlockSpec((2 * n, nt, _L), lambda i, t_pre: (0, 0, 0)),
            pl.BlockSpec((1, 2, _L), lambda i, t_pre: (t_pre[0], 0, 0)),
            pl.BlockSpec((1, 2, _L), lambda i, t_pre: (t_pre[0], 0, 0)),
        ],
        out_specs=pl.BlockSpec((_BN, _L), lambda i, t_pre: (i, 0)),
        scratch_shapes=[pltpu.VMEM((k, _L), jnp.bfloat16)],
    )
    out = pl.pallas_call(
        _tel_kernel,
        grid_spec=grid_spec,
        out_shape=jax.ShapeDtypeStruct((m, _L), jnp.float32),
        compiler_params=pltpu.CompilerParams(
            dimension_semantics=("arbitrary",)),
    )(t_arr, xt, wv, alpha, beta)

    return out.reshape(n_ent, nb, _L)


# speedup vs baseline: 1.0115x; 1.0115x over previous
import jax
import jax.numpy as jnp
from jax.experimental import pallas as pl
from jax.experimental.pallas import tpu as pltpu

_N = 250
_L = 128
_BN = 7424


def _clamp01(x):
    return jnp.minimum(jnp.maximum(x, 0.0), 1.0)


def _tel_kernel(t_ref, xt_ref, wv_ref, a_ref, b_ref, out_ref, wf_ref):
    @pl.when(pl.program_id(0) == 0)
    def _prep():
        a = jax.nn.sigmoid(a_ref[0, 0:1, :])
        b = jax.nn.sigmoid(b_ref[0, 0:1, :])

        w_t = wv_ref[:, t_ref[0], :]

        wh = w_t[:_N]
        wh = jnp.exp(wh - jnp.max(wh, axis=0, keepdims=True))
        wh = wh * (a / jnp.sum(wh, axis=0, keepdims=True))

        ws = w_t[_N:]
        ws = jnp.exp(ws - jnp.max(ws, axis=0, keepdims=True))
        ws = ws * (b / jnp.sum(ws, axis=0, keepdims=True))

        c = 1.0 - _clamp01(a + b)

        w_full = jnp.concatenate([wh, ws, c], axis=0)
        wf_ref[...] = w_full.astype(jnp.bfloat16)

    acc = jax.lax.dot_general(xt_ref[...].astype(jnp.bfloat16),
                              wf_ref[...],
                              (((0,), (0,)), ((), ())),
                              preferred_element_type=jnp.float32)
    out_ref[...] = _clamp01(acc)


def kernel(input, input_all, all_states, t, entity2id, flag, w, w_inv,
           weight, alpha, beta):
    n = _N
    n_ent = entity2id.shape[0]
    m = input.shape[0]
    k = input.shape[1]
    nb = m // n_ent
    nt = w.shape[0]

    xt = input.T
    wv = w.transpose(2, 0, 1)
    t_arr = jnp.asarray(t, jnp.int32).reshape(1)

    grid_spec = pltpu.PrefetchScalarGridSpec(
        num_scalar_prefetch=1,
        grid=(pl.cdiv(m, _BN),),
        in_specs=[
            pl.BlockSpec((k, _BN), lambda i, t_pre: (0, i)),
            pl.BlockSpec((2 * n, nt, _L), lambda i, t_pre: (0, 0, 0)),
            pl.BlockSpec((1, 2, _L), lambda i, t_pre: (t_pre[0], 0, 0)),
            pl.BlockSpec((1, 2, _L), lambda i, t_pre: (t_pre[0], 0, 0)),
        ],
        out_specs=pl.BlockSpec((_BN, _L), lambda i, t_pre: (i, 0)),
        scratch_shapes=[pltpu.VMEM((k, _L), jnp.bfloat16)],
    )
    out = pl.pallas_call(
        _tel_kernel,
        grid_spec=grid_spec,
        out_shape=jax.ShapeDtypeStruct((m, _L), jnp.float32),
        compiler_params=pltpu.CompilerParams(
            dimension_semantics=("arbitrary",)),
    )(t_arr, xt, wv, alpha, beta)

    return out.reshape(n_ent, nb, _L)


# speedup vs baseline: 1.0273x; 1.0156x over previous
import jax
import jax.numpy as jnp
from jax.experimental import pallas as pl
from jax.experimental.pallas import tpu as pltpu

_N = 250
_L = 128
_BN = 7168


def _clamp01(x):
    return jnp.minimum(jnp.maximum(x, 0.0), 1.0)


def _tel_kernel(t_ref, xt_ref, wv_ref, a_ref, b_ref, out_ref, wf_ref):
    @pl.when(pl.program_id(0) == 0)
    def _prep():
        a = jax.nn.sigmoid(a_ref[0, 0:1, :])
        b = jax.nn.sigmoid(b_ref[0, 0:1, :])

        w_t = wv_ref[:, t_ref[0], :]

        wh = w_t[:_N]
        wh = jnp.exp(wh - jnp.max(wh, axis=0, keepdims=True))
        wh = wh * (a / jnp.sum(wh, axis=0, keepdims=True))

        ws = w_t[_N:]
        ws = jnp.exp(ws - jnp.max(ws, axis=0, keepdims=True))
        ws = ws * (b / jnp.sum(ws, axis=0, keepdims=True))

        c = 1.0 - _clamp01(a + b)

        w_full = jnp.concatenate([wh, ws, c], axis=0)
        wf_ref[...] = w_full.astype(jnp.bfloat16)

    acc = jax.lax.dot_general(xt_ref[...].astype(jnp.bfloat16),
                              wf_ref[...],
                              (((0,), (0,)), ((), ())),
                              preferred_element_type=jnp.float32)
    out_ref[...] = _clamp01(acc)


def kernel(input, input_all, all_states, t, entity2id, flag, w, w_inv,
           weight, alpha, beta):
    n = _N
    n_ent = entity2id.shape[0]
    m = input.shape[0]
    k = input.shape[1]
    nb = m // n_ent
    nt = w.shape[0]

    xt = input.T
    wv = w.transpose(2, 0, 1)
    t_arr = jnp.asarray(t, jnp.int32).reshape(1)

    grid_spec = pltpu.PrefetchScalarGridSpec(
        num_scalar_prefetch=1,
        grid=(pl.cdiv(m, _BN),),
        in_specs=[
            pl.BlockSpec((k, _BN), lambda i, t_pre: (0, i)),
            pl.BlockSpec((2 * n, nt, _L), lambda i, t_pre: (0, 0, 0)),
            pl.BlockSpec((1, 2, _L), lambda i, t_pre: (t_pre[0], 0, 0)),
            pl.BlockSpec((1, 2, _L), lambda i, t_pre: (t_pre[0], 0, 0)),
        ],
        out_specs=pl.BlockSpec((_BN, _L), lambda i, t_pre: (i, 0)),
        scratch_shapes=[pltpu.VMEM((k, _L), jnp.bfloat16)],
    )
    out = pl.pallas_call(
        _tel_kernel,
        grid_spec=grid_spec,
        out_shape=jax.ShapeDtypeStruct((m, _L), jnp.float32),
        compiler_params=pltpu.CompilerParams(
            dimension_semantics=("arbitrary",)),
    )(t_arr, xt, wv, alpha, beta)

    return out.reshape(n_ent, nb, _L)
